# Initial kernel scaffold; baseline (speedup 1.0000x reference)
import jax
import jax.numpy as jnp
from jax.experimental import pallas as pl


def kernel(x, edge_index, W_pre, b_pre, Wg0, bg0, Wg1, bg1, Wg2, bg2, W_post, b_post):
    raise NotImplementedError("write your pallas kernel here")


# speedup vs baseline: 2.6542x; 2.6542x over previous
import functools

import jax
import jax.numpy as jnp
from jax import lax
from jax.experimental import pallas as pl
from jax.experimental.pallas import tpu as pltpu
from jax.experimental.pallas import tpu_sc as plsc

NC = 2
NS = 16
L = 16

N_PAD = 10240
CHUNKS = 4
CW = 128
KB = 128
ROWS_PER_TILE = N_PAD // NS
OUT_ROWS_PER_WORKER = N_PAD // (NC * NS)


def _fill_vmem_2d(ref, nrows, val):
  row16 = jnp.full((L,), val, jnp.float32)
  ncol_blk = ref.shape[1] // L

  def body(i, _):
    for cb in range(ncol_blk):
      ref[i, pl.ds(cb * L, L)] = row16
    return 0

  lax.fori_loop(0, nrows, body, 0)


def _sc_degree_body(src3, deg, degacc, vals, idxb):
  c = lax.axis_index("c")
  s = lax.axis_index("s")
  nb = idxb.shape[0]

  _fill_vmem_2d(vals, 128, 0.0)
  for k in range(ROWS_PER_TILE // 128):
    pltpu.sync_copy(vals, degacc.at[pl.ds(s * ROWS_PER_TILE + k * 128, 128)])
  _fill_vmem_2d(vals, 128, 1.0)
  plsc.subcore_barrier()

  pltpu.sync_copy(src3.at[s], idxb)

  def edge_body(j, _):
    pltpu.sync_copy(vals, degacc.at[idxb.at[j]], add=True)
    return 0

  lax.fori_loop(0, nb, edge_body, 0)
  plsc.subcore_barrier()

  w = c * NS + s
  base = w * OUT_ROWS_PER_WORKER
  pltpu.sync_copy(degacc.at[pl.ds(base, OUT_ROWS_PER_WORKER)],
                  deg.at[pl.ds(base, OUT_ROWS_PER_WORKER)])


def _sc_aggregate_body(h4, dst3, src3, agg4, acc, didx, sidx, rows, sem):
  c = lax.axis_index("c")
  s = lax.axis_index("s")
  nb = didx.shape[0]

  pltpu.sync_copy(dst3.at[s], didx)
  pltpu.sync_copy(src3.at[s], sidx)

  for chunk in range(CHUNKS):
    active = c == (chunk // 2)

    @pl.when(active)
    def _():
      _fill_vmem_2d(rows, 128, 0.0)
      for k in range(ROWS_PER_TILE // 128):
        pltpu.sync_copy(rows, acc.at[pl.ds(s * ROWS_PER_TILE + k * 128, 128)])

    plsc.subcore_barrier()

    @pl.when(active)
    def _():
      h_t = h4.at[chunk]

      def edge_body(j, _):
        pltpu.async_copy(h_t.at[didx.at[j]], rows, sem).wait()
        pltpu.sync_copy(rows, acc.at[sidx.at[j]], add=True)
        return 0

      lax.fori_loop(0, nb, edge_body, 0)

    plsc.subcore_barrier()

    @pl.when(active)
    def _():
      pltpu.sync_copy(
          acc.at[pl.ds(s * ROWS_PER_TILE, ROWS_PER_TILE)],
          agg4.at[chunk].at[pl.ds(s * ROWS_PER_TILE, ROWS_PER_TILE)],
      )

    plsc.subcore_barrier()


def _sc_degree(src3):
  nb = src3.shape[1]
  mesh = plsc.VectorSubcoreMesh(
      core_axis_name="c", subcore_axis_name="s", num_cores=NC, num_subcores=NS)
  return pl.kernel(
      _sc_degree_body,
      out_type=jax.ShapeDtypeStruct((N_PAD, CW), jnp.float32),
      mesh=mesh,
      scratch_types=[
          pltpu.VMEM_SHARED((N_PAD, CW), jnp.float32),
          pltpu.VMEM((128, CW), jnp.float32),
          pltpu.VMEM((nb, KB), jnp.int32),
      ],
  )(src3)


def _sc_aggregate(h4, dst3, src3):
  nb = dst3.shape[1]
  mesh = plsc.VectorSubcoreMesh(
      core_axis_name="c", subcore_axis_name="s", num_cores=NC, num_subcores=NS)
  return pl.kernel(
      _sc_aggregate_body,
      out_type=jax.ShapeDtypeStruct((CHUNKS, N_PAD, CW), jnp.float32),
      mesh=mesh,
      scratch_types=[
          pltpu.VMEM_SHARED((N_PAD, CW), jnp.float32),
          pltpu.VMEM((nb, KB), jnp.int32),
          pltpu.VMEM((nb, KB), jnp.int32),
          pltpu.VMEM((KB, CW), jnp.float32),
          pltpu.SemaphoreType.DMA,
      ],
  )(h4, dst3, src3)


RB = 1024


def _tc_pre_body(x_ref, w_ref, b_ref, out_ref):
  xw = jnp.dot(x_ref[...], w_ref[...], preferred_element_type=jnp.float32)
  out_ref[0] = jnp.maximum(xw + b_ref[0], 0.0)


def _tc_pre(x_p, w, b4):
  d_in = x_p.shape[1]
  grid = (CHUNKS, N_PAD // RB)
  return pl.pallas_call(
      _tc_pre_body,
      grid=grid,
      in_specs=[
          pl.BlockSpec((RB, d_in), lambda co, rb: (rb, 0)),
          pl.BlockSpec((d_in, CW), lambda co, rb: (0, co)),
          pl.BlockSpec((1, 1, CW), lambda co, rb: (co, 0, 0)),
      ],
      out_specs=pl.BlockSpec((1, RB, CW), lambda co, rb: (co, rb, 0)),
      out_shape=jax.ShapeDtypeStruct((CHUNKS, N_PAD, CW), jnp.float32),
  )(x_p, w, b4)


def _tc_layer_body(agg_ref, invd_ref, w_ref, b_ref, out_ref):
  ki = pl.program_id(2)

  @pl.when(ki == 0)
  def _():
    out_ref[...] = jnp.zeros_like(out_ref)

  scaled = agg_ref[0] * (1.0 / jnp.maximum(invd_ref[...], 1.0))
  out_ref[0] += jnp.dot(scaled, w_ref[0], preferred_element_type=jnp.float32)

  @pl.when(ki == CHUNKS - 1)
  def _():
    out_ref[0] = jnp.maximum(out_ref[0] + b_ref[0], 0.0)


def _tc_layer(agg4, invd2, w4, b4):
  grid = (CHUNKS, N_PAD // RB, CHUNKS)
  return pl.pallas_call(
      _tc_layer_body,
      grid=grid,
      in_specs=[
          pl.BlockSpec((1, RB, CW), lambda co, rb, ki: (ki, rb, 0)),
          pl.BlockSpec((RB, 1), lambda co, rb, ki: (rb, 0)),
          pl.BlockSpec((1, CW, CW), lambda co, rb, ki: (ki, 0, co)),
          pl.BlockSpec((1, 1, CW), lambda co, rb, ki: (co, 0, 0)),
      ],
      out_specs=pl.BlockSpec((1, RB, CW), lambda co, rb, ki: (co, rb, 0)),
      out_shape=jax.ShapeDtypeStruct((CHUNKS, N_PAD, CW), jnp.float32),
  )(agg4, invd2, w4, b4)


def _tc_post_body(h_ref, w_ref, b_ref, out_ref):
  ki = pl.program_id(1)

  @pl.when(ki == 0)
  def _():
    out_ref[...] = jnp.zeros_like(out_ref)

  out_ref[...] += jnp.dot(h_ref[0], w_ref[0], preferred_element_type=jnp.float32)

  @pl.when(ki == CHUNKS - 1)
  def _():
    out_ref[...] += b_ref[...]


def _tc_post(h4, w4, b2):
  d_out = w4.shape[2]
  grid = (N_PAD // RB, CHUNKS)
  return pl.pallas_call(
      _tc_post_body,
      grid=grid,
      in_specs=[
          pl.BlockSpec((1, RB, CW), lambda rb, ki: (ki, rb, 0)),
          pl.BlockSpec((1, CW, d_out), lambda rb, ki: (ki, 0, 0)),
          pl.BlockSpec((1, d_out), lambda rb, ki: (0, 0)),
      ],
      out_specs=pl.BlockSpec((RB, d_out), lambda rb, ki: (rb, 0)),
      out_shape=jax.ShapeDtypeStruct((N_PAD, d_out), jnp.float32),
  )(h4, w4, b2)


@jax.jit
def kernel(x, edge_index, W_pre, b_pre, Wg0, bg0, Wg1, bg1, Wg2, bg2,
           W_post, b_post):
  n, d_in = x.shape
  e = edge_index.shape[1]
  d_h = W_pre.shape[1]
  d_out = W_post.shape[1]
  assert d_h == CHUNKS * CW

  e_tile = -(-e // (NS * KB)) * KB
  e_pad = NS * e_tile
  src = edge_index[0]
  dst = edge_index[1]
  pad_idx = jnp.full((e_pad - e,), N_PAD - 1, jnp.int32)
  src3 = jnp.concatenate([src, pad_idx]).reshape(NS, e_tile // KB, KB)
  dst3 = jnp.concatenate([dst, pad_idx]).reshape(NS, e_tile // KB, KB)
  x_p = jnp.pad(x, ((0, N_PAD - n), (0, 0)))

  deg = _sc_degree(src3)
  invd2 = deg[:, :1]

  h4 = _tc_pre(x_p, W_pre, b_pre.reshape(CHUNKS, 1, CW))
  for w, b in ((Wg0, bg0), (Wg1, bg1), (Wg2, bg2)):
    agg4 = _sc_aggregate(h4, dst3, src3)
    h4 = _tc_layer(agg4, invd2, w.reshape(CHUNKS, CW, d_h),
                   b.reshape(CHUNKS, 1, CW))
  logits = _tc_post(h4, W_post.reshape(CHUNKS, CW, d_out),
                    b_post.reshape(1, d_out))
  return logits[:n]
